# Initial kernel scaffold; baseline (speedup 1.0000x reference)
import jax
import jax.numpy as jnp
from jax.experimental import pallas as pl


def kernel(x, edge_index, W1, b1, W2, b2, Wfc, bfc):
    raise NotImplementedError("write your pallas kernel here")


# speedup vs baseline: 20.0320x; 20.0320x over previous
import functools

import jax
import jax.numpy as jnp
from jax import lax
from jax.experimental import pallas as pl
from jax.experimental.pallas import tpu as pltpu
from jax.experimental.pallas import tpu_sc as plsc

N = 10000
E = 160000
FEAT = 256
HID = 256
OUT = 40

NP = 10240
HALF = 128
NC = 2
NS = 16
CHUNK = 128
ACH = 128
EP = 163840
ECH_T = EP // NS // ACH
PH = 2
PCH = ECH_T // PH
DCH_T = EP // (NC * NS) // CHUNK
RPT = NP // NS
ROWB = 1024


def _mesh():
    return plsc.VectorSubcoreMesh(core_axis_name="c", subcore_axis_name="s")


def _deg_body(dst_hbm, out, dst_v, ones_v, zero_v, acc):
    c = lax.axis_index("c")
    s = lax.axis_index("s")
    wid = c * NS + s

    def zloop(i, _):
        zero_v[pl.ds(i * 16, 16)] = jnp.zeros((16,), jnp.float32)
        return 0

    lax.fori_loop(0, RPT // 16, zloop, 0)

    def oloop(i, _):
        ones_v[pl.ds(i * 16, 16)] = jnp.ones((16,), jnp.float32)
        return 0

    lax.fori_loop(0, CHUNK // 16, oloop, 0)

    pltpu.sync_copy(dst_hbm.at[wid], dst_v)
    pltpu.sync_copy(zero_v, acc.at[pl.ds(s * RPT, RPT)])
    plsc.subcore_barrier()

    def body(j, _):
        pltpu.sync_copy(ones_v, acc.at[dst_v.at[j]], add=True)
        return 0

    lax.fori_loop(0, DCH_T, body, 0)
    plsc.subcore_barrier()
    pltpu.sync_copy(acc.at[pl.ds(s * RPT, RPT)], out.at[c, pl.ds(s * RPT, RPT)])


_deg_kernel = functools.partial(
    pl.kernel,
    out_type=jax.ShapeDtypeStruct((NC, NP), jnp.float32),
    mesh=_mesh(),
    scratch_types=[
        pltpu.VMEM((DCH_T, CHUNK), jnp.int32),
        pltpu.VMEM((CHUNK,), jnp.float32),
        pltpu.VMEM((RPT,), jnp.float32),
        pltpu.VMEM_SHARED((NP,), jnp.float32),
    ],
)(_deg_body)


def _agg_body(ga_hbm, gb_hbm, src_hbm, dst_hbm, outa, outb,
              src_v, dst_v, rows, acc, sem0, sem1):
    c = lax.axis_index("c")
    s = lax.axis_index("s")
    sems = (sem0, sem1)

    def run(g_hbm, out_hbm):
        pltpu.sync_copy(g_hbm.at[pl.ds(s * RPT, RPT)], acc.at[pl.ds(s * RPT, RPT)])
        plsc.subcore_barrier()
        for p in range(PH):
            pltpu.sync_copy(src_hbm.at[s, pl.ds(p * PCH, PCH)], src_v)
            pltpu.sync_copy(dst_hbm.at[s, pl.ds(p * PCH, PCH)], dst_v)
            for b in range(2):
                pltpu.async_copy(g_hbm.at[src_v.at[b]], rows.at[b], sems[b])

            def body(i, _):
                j = i * 2
                for b in range(2):
                    pltpu.make_async_copy(
                        g_hbm.at[pl.ds(0, ACH)], rows.at[b], sems[b]).wait()
                    pltpu.sync_copy(rows.at[b], acc.at[dst_v.at[j + b]], add=True)

                    @pl.when(j + b + 2 < PCH)
                    def _():
                        pltpu.async_copy(
                            g_hbm.at[src_v.at[j + b + 2]], rows.at[b], sems[b])
                return 0

            lax.fori_loop(0, PCH // 2, body, 0)
        plsc.subcore_barrier()
        pltpu.sync_copy(acc.at[pl.ds(s * RPT, RPT)], out_hbm.at[pl.ds(s * RPT, RPT)])

    @pl.when(c == 0)
    def _():
        run(ga_hbm, outa)

    @pl.when(c == 1)
    def _():
        run(gb_hbm, outb)


_agg_kernel = functools.partial(
    pl.kernel,
    out_type=(jax.ShapeDtypeStruct((NP, HALF), jnp.float32),
              jax.ShapeDtypeStruct((NP, HALF), jnp.float32)),
    mesh=_mesh(),
    scratch_types=[
        pltpu.VMEM((PCH, ACH), jnp.int32),
        pltpu.VMEM((PCH, ACH), jnp.int32),
        pltpu.VMEM((2, ACH, HALF), jnp.float32),
        pltpu.VMEM_SHARED((NP, HALF), jnp.float32),
        pltpu.SemaphoreType.DMA,
        pltpu.SemaphoreType.DMA,
    ],
)(_agg_body)


def _mm1_body(x_ref, w_ref, dega_ref, degb_ref, ga_ref, gb_ref, dinv_ref):
    dinv = lax.rsqrt(dega_ref[...] + degb_ref[...] + 1.0)
    h = jnp.dot(x_ref[...], w_ref[...], preferred_element_type=jnp.float32)
    g = h * dinv[:, None]
    ga_ref[...] = g[:, :HALF]
    gb_ref[...] = g[:, HALF:]
    dinv_ref[...] = dinv


def _mm2_body(acca_ref, accb_ref, dinv_ref, b_ref, w_ref, ga_ref, gb_ref):
    dinv = dinv_ref[...]
    acc = jnp.concatenate([acca_ref[...], accb_ref[...]], axis=1)
    h = jnp.maximum(acc * dinv[:, None] + b_ref[...][None, :], 0.0)
    g = jnp.dot(h, w_ref[...], preferred_element_type=jnp.float32) * dinv[:, None]
    ga_ref[...] = g[:, :HALF]
    gb_ref[...] = g[:, HALF:]


def _head_body(acca_ref, accb_ref, dinv_ref, b_ref, w_ref, bfc_ref, out_ref):
    dinv = dinv_ref[...]
    acc = jnp.concatenate([acca_ref[...], accb_ref[...]], axis=1)
    e = jnp.maximum(acc * dinv[:, None] + b_ref[...][None, :], 0.0)
    out_ref[...] = (jnp.dot(e, w_ref[...], preferred_element_type=jnp.float32)
                    + bfc_ref[...][None, :])


_GRID = (NP // ROWB,)


def _row_spec(cols):
    return pl.BlockSpec((ROWB, cols), lambda i: (i, 0))


def _vec_spec():
    return pl.BlockSpec((ROWB,), lambda i: (i,))


def _full_spec(r, c):
    return pl.BlockSpec((r, c), lambda i: (0, 0))


def _full1_spec(n):
    return pl.BlockSpec((n,), lambda i: (0,))


_mm1_call = pl.pallas_call(
    _mm1_body,
    grid=_GRID,
    in_specs=[_row_spec(FEAT), _full_spec(FEAT, HID), _vec_spec(), _vec_spec()],
    out_specs=(_row_spec(HALF), _row_spec(HALF), _vec_spec()),
    out_shape=(jax.ShapeDtypeStruct((NP, HALF), jnp.float32),
               jax.ShapeDtypeStruct((NP, HALF), jnp.float32),
               jax.ShapeDtypeStruct((NP,), jnp.float32)),
)

_mm2_call = pl.pallas_call(
    _mm2_body,
    grid=_GRID,
    in_specs=[_row_spec(HALF), _row_spec(HALF), _vec_spec(),
              _full1_spec(HID), _full_spec(HID, HID)],
    out_specs=(_row_spec(HALF), _row_spec(HALF)),
    out_shape=(jax.ShapeDtypeStruct((NP, HALF), jnp.float32),
               jax.ShapeDtypeStruct((NP, HALF), jnp.float32)),
)

_head_call = pl.pallas_call(
    _head_body,
    grid=_GRID,
    in_specs=[_row_spec(HALF), _row_spec(HALF), _vec_spec(),
              _full1_spec(HID), _full_spec(HID, HALF), _full1_spec(HALF)],
    out_specs=_row_spec(HALF),
    out_shape=jax.ShapeDtypeStruct((NP, HALF), jnp.float32),
)


def kernel(x, edge_index, W1, b1, W2, b2, Wfc, bfc):
    xp = jnp.zeros((NP, FEAT), jnp.float32).at[:N].set(x)
    src = edge_index[0]
    dst = edge_index[1]
    npad = EP - E
    pad_ids = (N + jnp.arange(npad, dtype=jnp.int32) % (NP - N)).astype(jnp.int32)
    srcp = jnp.concatenate([src, pad_ids])
    dstp = jnp.concatenate([dst, pad_ids])
    src_t = srcp.reshape(NS, ECH_T, ACH)
    dst_t = dstp.reshape(NS, ECH_T, ACH)
    dst_d = dstp.reshape(NC * NS, DCH_T, CHUNK)
    wfc_p = jnp.zeros((HID, HALF), jnp.float32).at[:, :OUT].set(Wfc)
    bfc_p = jnp.zeros((HALF,), jnp.float32).at[:OUT].set(bfc)

    deg = _deg_kernel(dst_d)
    ga1, gb1, dinv = _mm1_call(xp, W1, deg[0], deg[1])
    acc1a, acc1b = _agg_kernel(ga1, gb1, src_t, dst_t)
    ga2, gb2 = _mm2_call(acc1a, acc1b, dinv, b1, W2)
    acc2a, acc2b = _agg_kernel(ga2, gb2, src_t, dst_t)
    logits_p = _head_call(acc2a, acc2b, dinv, b2, wfc_p, bfc_p)
    return logits_p[:N, :OUT]


# speedup vs baseline: 20.1073x; 1.0038x over previous
import functools

import jax
import jax.numpy as jnp
from jax import lax
from jax.experimental import pallas as pl
from jax.experimental.pallas import tpu as pltpu
from jax.experimental.pallas import tpu_sc as plsc

N = 10000
E = 160000
FEAT = 256
HID = 256
OUT = 40

NP = 10240
HALF = 128
NC = 2
NS = 16
CHUNK = 128
ACH = 128
EP = 163840
ECH_T = EP // NS // ACH
PH = 2
PCH = ECH_T // PH
DCH_T = EP // (NC * NS) // CHUNK
RPT = NP // NS
ROWB = 1024


def _mesh():
    return plsc.VectorSubcoreMesh(core_axis_name="c", subcore_axis_name="s")


def _deg_body(dst_hbm, out, dst_v, ones_v, zero_v, acc):
    c = lax.axis_index("c")
    s = lax.axis_index("s")
    wid = c * NS + s

    def zloop(i, _):
        zero_v[pl.ds(i * 16, 16)] = jnp.zeros((16,), jnp.float32)
        return 0

    lax.fori_loop(0, RPT // 16, zloop, 0)

    def oloop(i, _):
        ones_v[pl.ds(i * 16, 16)] = jnp.ones((16,), jnp.float32)
        return 0

    lax.fori_loop(0, CHUNK // 16, oloop, 0)

    pltpu.sync_copy(dst_hbm.at[wid], dst_v)
    pltpu.sync_copy(zero_v, acc.at[pl.ds(s * RPT, RPT)])
    plsc.subcore_barrier()

    def body(j, _):
        pltpu.sync_copy(ones_v, acc.at[dst_v.at[j]], add=True)
        return 0

    lax.fori_loop(0, DCH_T, body, 0)
    plsc.subcore_barrier()
    pltpu.sync_copy(acc.at[pl.ds(s * RPT, RPT)], out.at[c, pl.ds(s * RPT, RPT)])


_deg_kernel = functools.partial(
    pl.kernel,
    out_type=jax.ShapeDtypeStruct((NC, NP), jnp.float32),
    mesh=_mesh(),
    scratch_types=[
        pltpu.VMEM((DCH_T, CHUNK), jnp.int32),
        pltpu.VMEM((CHUNK,), jnp.float32),
        pltpu.VMEM((RPT,), jnp.float32),
        pltpu.VMEM_SHARED((NP,), jnp.float32),
    ],
)(_deg_body)


def _agg_body(ga_hbm, gb_hbm, src_hbm, dst_hbm, outa, outb,
              src_v, dst_v, rows, acc, sem0, sem1):
    c = lax.axis_index("c")
    s = lax.axis_index("s")
    sems = (sem0, sem1)

    def run(g_hbm, out_hbm):
        pltpu.sync_copy(g_hbm.at[pl.ds(s * RPT, RPT)], acc.at[pl.ds(s * RPT, RPT)])
        plsc.subcore_barrier()
        for p in range(PH):
            pltpu.sync_copy(src_hbm.at[s, pl.ds(p * PCH, PCH)], src_v)
            pltpu.sync_copy(dst_hbm.at[s, pl.ds(p * PCH, PCH)], dst_v)
            for b in range(2):
                pltpu.async_copy(g_hbm.at[src_v.at[b]], rows.at[b], sems[b])

            def body(i, _):
                j = i * 2
                for b in range(2):
                    pltpu.make_async_copy(
                        g_hbm.at[pl.ds(0, ACH)], rows.at[b], sems[b]).wait()
                    pltpu.sync_copy(rows.at[b], acc.at[dst_v.at[j + b]], add=True)

                    @pl.when(j + b + 2 < PCH)
                    def _():
                        pltpu.async_copy(
                            g_hbm.at[src_v.at[j + b + 2]], rows.at[b], sems[b])
                return 0

            lax.fori_loop(0, PCH // 2, body, 0)
        plsc.subcore_barrier()
        pltpu.sync_copy(acc.at[pl.ds(s * RPT, RPT)], out_hbm.at[pl.ds(s * RPT, RPT)])

    @pl.when(c == 0)
    def _():
        run(ga_hbm, outa)

    @pl.when(c == 1)
    def _():
        run(gb_hbm, outb)


_agg_kernel = functools.partial(
    pl.kernel,
    out_type=(jax.ShapeDtypeStruct((NP, HALF), jnp.float32),
              jax.ShapeDtypeStruct((NP, HALF), jnp.float32)),
    mesh=_mesh(),
    scratch_types=[
        pltpu.VMEM((PCH, ACH), jnp.int32),
        pltpu.VMEM((PCH, ACH), jnp.int32),
        pltpu.VMEM((2, ACH, HALF), jnp.float32),
        pltpu.VMEM_SHARED((NP, HALF), jnp.float32),
        pltpu.SemaphoreType.DMA,
        pltpu.SemaphoreType.DMA,
    ],
)(_agg_body)


def _mm1_body(x_ref, w_ref, dega_ref, degb_ref, ga_ref, gb_ref, dinv_ref):
    dinv = lax.rsqrt(dega_ref[...] + degb_ref[...] + 1.0)
    h = jnp.dot(x_ref[...], w_ref[...], preferred_element_type=jnp.float32)
    g = h * dinv[:, None]
    ga_ref[...] = g[:, :HALF]
    gb_ref[...] = g[:, HALF:]
    dinv_ref[...] = dinv


def _mm2_body(acca_ref, accb_ref, dinv_ref, b_ref, w_ref, ga_ref, gb_ref):
    dinv = dinv_ref[...]
    acc = jnp.concatenate([acca_ref[...], accb_ref[...]], axis=1)
    h = jnp.maximum(acc * dinv[:, None] + b_ref[...][None, :], 0.0)
    g = jnp.dot(h, w_ref[...], preferred_element_type=jnp.float32) * dinv[:, None]
    ga_ref[...] = g[:, :HALF]
    gb_ref[...] = g[:, HALF:]


def _head_body(acca_ref, accb_ref, dinv_ref, b_ref, w_ref, bfc_ref, out_ref):
    dinv = dinv_ref[...]
    acc = jnp.concatenate([acca_ref[...], accb_ref[...]], axis=1)
    e = jnp.maximum(acc * dinv[:, None] + b_ref[...][None, :], 0.0)
    out_ref[...] = (jnp.dot(e, w_ref[...], preferred_element_type=jnp.float32)
                    + bfc_ref[...][None, :])


_GRID = (NP // ROWB,)


def _row_spec(cols):
    return pl.BlockSpec((ROWB, cols), lambda i: (i, 0))


def _vec_spec():
    return pl.BlockSpec((ROWB,), lambda i: (i,))


def _full_spec(r, c):
    return pl.BlockSpec((r, c), lambda i: (0, 0))


def _full1_spec(n):
    return pl.BlockSpec((n,), lambda i: (0,))


_mm1_call = pl.pallas_call(
    _mm1_body,
    grid=_GRID,
    in_specs=[_row_spec(FEAT), _full_spec(FEAT, HID), _vec_spec(), _vec_spec()],
    out_specs=(_row_spec(HALF), _row_spec(HALF), _vec_spec()),
    out_shape=(jax.ShapeDtypeStruct((NP, HALF), jnp.float32),
               jax.ShapeDtypeStruct((NP, HALF), jnp.float32),
               jax.ShapeDtypeStruct((NP,), jnp.float32)),
)

_mm2_call = pl.pallas_call(
    _mm2_body,
    grid=_GRID,
    in_specs=[_row_spec(HALF), _row_spec(HALF), _vec_spec(),
              _full1_spec(HID), _full_spec(HID, HID)],
    out_specs=(_row_spec(HALF), _row_spec(HALF)),
    out_shape=(jax.ShapeDtypeStruct((NP, HALF), jnp.float32),
               jax.ShapeDtypeStruct((NP, HALF), jnp.float32)),
)

_head_call = pl.pallas_call(
    _head_body,
    grid=_GRID,
    in_specs=[_row_spec(HALF), _row_spec(HALF), _vec_spec(),
              _full1_spec(HID), _full_spec(HID, OUT), _full1_spec(OUT)],
    out_specs=_row_spec(OUT),
    out_shape=jax.ShapeDtypeStruct((N, OUT), jnp.float32),
)


def kernel(x, edge_index, W1, b1, W2, b2, Wfc, bfc):
    src = edge_index[0]
    dst = edge_index[1]
    npad = EP - E
    pad_ids = (N + jnp.arange(npad, dtype=jnp.int32) % (NP - N)).astype(jnp.int32)
    srcp = jnp.concatenate([src, pad_ids])
    dstp = jnp.concatenate([dst, pad_ids])
    src_t = srcp.reshape(NS, ECH_T, ACH)
    dst_t = dstp.reshape(NS, ECH_T, ACH)
    dst_d = dstp.reshape(NC * NS, DCH_T, CHUNK)

    deg = _deg_kernel(dst_d)
    ga1, gb1, dinv = _mm1_call(x, W1, deg[0], deg[1])
    acc1a, acc1b = _agg_kernel(ga1, gb1, src_t, dst_t)
    ga2, gb2 = _mm2_call(acc1a, acc1b, dinv, b1, W2)
    acc2a, acc2b = _agg_kernel(ga2, gb2, src_t, dst_t)
    return _head_call(acc2a, acc2b, dinv, b2, Wfc, bfc)
